# baseline (speedup 1.0000x reference)
import numpy as np
import jax
import jax.numpy as jnp
from jax import lax
from jax.experimental import pallas as pl
from jax.experimental.pallas import tpu as pltpu
from jax.experimental.pallas import tpu_sc as plsc

NUM_WORDS = 100000
EMB_DIM = 128
NUM_SAMPLES = 8192
BATCH = 16384
TINY = 1e-13
INV_LOG_NW = float(1.0 / np.log(NUM_WORDS + 1))
LOG2E = float(np.log2(np.e))
LN2 = float(np.log(2.0))

BM = 1024

_NC = 2
_NS = 16
_NTILES = _NC * _NS
_TGT_PER = BATCH // _NTILES
_SMP_PER = NUM_SAMPLES // _NTILES


def _sc_gather_body(w_hbm, b_hbm, tgt_hbm, smp_hbm,
                    tw_out, sw_out, tb_out, sb_out,
                    idxt_v, idxs_v, trows_v, srows_v, tbias_v, sbias_v, sem):
    c = lax.axis_index("c")
    s = lax.axis_index("s")
    wid = s * _NC + c
    tbase = wid * _TGT_PER
    sbase = wid * _SMP_PER
    pltpu.sync_copy(tgt_hbm.at[pl.ds(tbase, _TGT_PER)], idxt_v)
    pltpu.sync_copy(smp_hbm.at[pl.ds(sbase, _SMP_PER)], idxs_v)
    cps = [
        pltpu.async_copy(w_hbm.at[idxt_v], trows_v, sem),
        pltpu.async_copy(w_hbm.at[idxs_v], srows_v, sem),
        pltpu.async_copy(b_hbm.at[idxt_v], tbias_v, sem),
        pltpu.async_copy(b_hbm.at[idxs_v], sbias_v, sem),
    ]
    for cp in cps:
        cp.wait()
    pltpu.sync_copy(trows_v, tw_out.at[pl.ds(tbase, _TGT_PER)])
    pltpu.sync_copy(srows_v, sw_out.at[pl.ds(sbase, _SMP_PER)])
    pltpu.sync_copy(tbias_v, tb_out.at[pl.ds(tbase, _TGT_PER)])
    pltpu.sync_copy(sbias_v, sb_out.at[pl.ds(sbase, _SMP_PER)])


def _sc_gather(softmax_w, softmax_b, targets, sampled_ids):
    mesh = plsc.VectorSubcoreMesh(core_axis_name="c", subcore_axis_name="s")
    f = pl.kernel(
        _sc_gather_body,
        out_type=(
            jax.ShapeDtypeStruct((BATCH, EMB_DIM), jnp.float32),
            jax.ShapeDtypeStruct((NUM_SAMPLES, EMB_DIM), jnp.float32),
            jax.ShapeDtypeStruct((BATCH,), jnp.float32),
            jax.ShapeDtypeStruct((NUM_SAMPLES,), jnp.float32),
        ),
        mesh=mesh,
        scratch_types=(
            pltpu.VMEM((_TGT_PER,), jnp.int32),
            pltpu.VMEM((_SMP_PER,), jnp.int32),
            pltpu.VMEM((_TGT_PER, EMB_DIM), jnp.float32),
            pltpu.VMEM((_SMP_PER, EMB_DIM), jnp.float32),
            pltpu.VMEM((_TGT_PER,), jnp.float32),
            pltpu.VMEM((_SMP_PER,), jnp.float32),
            pltpu.SemaphoreType.DMA,
        ),
    )
    return f(softmax_w, softmax_b, targets, sampled_ids)


_VOCAB_PAD = 100096


_ZSL = _VOCAB_PAD // _NS
_SMP_PER_SC = NUM_SAMPLES // _NS
_TGT_PER_SC = BATCH // _NTILES


def _sc_member_body(zeros_hbm, ones_hbm, smp_hbm, tgt_hbm,
                    flags_out, b0_hbm, b1_hbm,
                    zbuf, idx_v, val_v, fval_v, sem):
    c = lax.axis_index("c")
    s = lax.axis_index("s")
    zbase = s * _ZSL
    pltpu.sync_copy(zeros_hbm.at[pl.ds(0, _ZSL)], zbuf)

    @pl.when(c == 0)
    def _():
        pltpu.sync_copy(zbuf, b0_hbm.at[pl.ds(zbase, _ZSL)])

    @pl.when(c == 1)
    def _():
        pltpu.sync_copy(zbuf, b1_hbm.at[pl.ds(zbase, _ZSL)])

    plsc.subcore_barrier()

    sbase = s * _SMP_PER_SC
    pltpu.sync_copy(smp_hbm.at[pl.ds(sbase, _SMP_PER_SC)], idx_v)
    pltpu.sync_copy(ones_hbm, val_v)

    @pl.when(c == 0)
    def _():
        pltpu.async_copy(val_v, b0_hbm.at[idx_v], sem).wait()

    @pl.when(c == 1)
    def _():
        pltpu.async_copy(val_v, b1_hbm.at[idx_v], sem).wait()

    plsc.subcore_barrier()

    tbase = c * (BATCH // _NC) + s * _TGT_PER_SC
    pltpu.sync_copy(tgt_hbm.at[pl.ds(tbase, _TGT_PER_SC)], idx_v)

    @pl.when(c == 0)
    def _():
        pltpu.async_copy(b0_hbm.at[idx_v], fval_v, sem).wait()

    @pl.when(c == 1)
    def _():
        pltpu.async_copy(b1_hbm.at[idx_v], fval_v, sem).wait()

    pltpu.sync_copy(fval_v, flags_out.at[pl.ds(tbase, _TGT_PER_SC)])


def _sc_member(sampled_ids, targets):
    mesh = plsc.VectorSubcoreMesh(core_axis_name="c", subcore_axis_name="s")
    f = pl.kernel(
        _sc_member_body,
        out_type=(
            jax.ShapeDtypeStruct((BATCH,), jnp.float32),
            jax.ShapeDtypeStruct((_VOCAB_PAD,), jnp.float32),
            jax.ShapeDtypeStruct((_VOCAB_PAD,), jnp.float32),
        ),
        mesh=mesh,
        scratch_types=(
            pltpu.VMEM((_ZSL,), jnp.float32),
            pltpu.VMEM((_SMP_PER_SC,), jnp.int32),
            pltpu.VMEM((_SMP_PER_SC,), jnp.float32),
            pltpu.VMEM((_TGT_PER_SC,), jnp.float32),
            pltpu.SemaphoreType.DMA,
        ),
    )
    zeros = jnp.zeros((_VOCAB_PAD,), jnp.float32)
    ones = jnp.ones((_SMP_PER_SC,), jnp.float32)
    flags, _, _ = f(zeros, ones, sampled_ids, targets)
    return flags


def _tc_loss_body(nt_ref, emb_ref, tw_ref, sw_ref, tb_ref, sb_ref,
                  tgt_ref, sid_ref, out_ref):
    step = pl.program_id(0)
    nt = nt_ref[0, 0]

    e2 = emb_ref[...] * LOG2E
    t_dot2 = jnp.sum(e2 * tw_ref[...], axis=1, keepdims=True)
    tgtf = tgt_ref[...].astype(jnp.float32)
    tp = jnp.log((tgtf + 2.0) / (tgtf + 1.0)) * INV_LOG_NW
    tec = 1.0 - jnp.exp(nt * jnp.log(1.0 - tp))
    t_logit2 = t_dot2 + tb_ref[...] * LOG2E - jnp.log2(tec + TINY)

    sidf = sid_ref[...].astype(jnp.float32)
    sp = jnp.log((sidf + 2.0) / (sidf + 1.0)) * INV_LOG_NW
    sec = 1.0 - jnp.exp(nt * jnp.log(1.0 - sp))
    adj2 = sb_ref[...] * LOG2E - jnp.log2(sec + TINY)

    logits2 = lax.dot_general(e2, sw_ref[...], (((1,), (1,)), ((), ())),
                              preferred_element_type=jnp.float32)
    x2 = (logits2 + adj2).astype(jnp.bfloat16)
    expl = jnp.exp2(x2)
    expl = jnp.where(sid_ref[...] == tgt_ref[...],
                     jnp.bfloat16(0.0), expl)
    ones = jnp.full((NUM_SAMPLES, 1), 1.0, jnp.bfloat16)
    ssum = lax.dot_general(expl, ones, (((1,), (0,)), ((), ())),
                           preferred_element_type=jnp.float32)
    den = ssum + jnp.exp2(t_logit2)
    partial = jnp.sum(jnp.log2(den) - t_logit2) * LN2

    @pl.when(step == 0)
    def _():
        out_ref[0, 0] = 0.0

    out_ref[0, 0] += partial


def _tc_loss(nt, emb, tw, sw, tb, sb, tgt, sid, interpret=False):
    grid = (BATCH // BM,)
    return pl.pallas_call(
        _tc_loss_body,
        grid=grid,
        in_specs=[
            pl.BlockSpec(memory_space=pltpu.SMEM),
            pl.BlockSpec((BM, EMB_DIM), lambda i: (i, 0)),
            pl.BlockSpec((BM, EMB_DIM), lambda i: (i, 0)),
            pl.BlockSpec((NUM_SAMPLES, EMB_DIM), lambda i: (0, 0)),
            pl.BlockSpec((BM, 1), lambda i: (i, 0)),
            pl.BlockSpec((1, NUM_SAMPLES), lambda i: (0, 0)),
            pl.BlockSpec((BM, 1), lambda i: (i, 0)),
            pl.BlockSpec((1, NUM_SAMPLES), lambda i: (0, 0)),
        ],
        out_specs=pl.BlockSpec(memory_space=pltpu.SMEM),
        out_shape=jax.ShapeDtypeStruct((1, 1), jnp.float32),
        interpret=interpret,
    )(nt, emb, tw, sw, tb, sb, tgt, sid)


def kernel(embeddings, targets, softmax_w, softmax_b, sampled_ids, num_tries):
    tw, sw, tb, sb = _sc_gather(softmax_w, softmax_b, targets, sampled_ids)
    nt = jnp.asarray(num_tries, jnp.float32).reshape(1, 1)
    loss = _tc_loss(
        nt, embeddings, tw, sw,
        tb.reshape(BATCH, 1), sb.reshape(1, NUM_SAMPLES),
        targets.reshape(BATCH, 1), sampled_ids.reshape(1, NUM_SAMPLES),
    )
    return loss[0, 0]


# speedup vs baseline: 1.1125x; 1.1125x over previous
import numpy as np
import jax
import jax.numpy as jnp
from jax import lax
from jax.experimental import pallas as pl
from jax.experimental.pallas import tpu as pltpu
from jax.experimental.pallas import tpu_sc as plsc

NUM_WORDS = 100000
EMB_DIM = 128
NUM_SAMPLES = 8192
BATCH = 16384
TINY = 1e-13
INV_LOG_NW = float(1.0 / np.log(NUM_WORDS + 1))
LOG2E = float(np.log2(np.e))
LN2 = float(np.log(2.0))

BM = 1024

_NC = 2
_NS = 16
_NTILES = _NC * _NS
_TGT_PER = BATCH // _NTILES
_SMP_PER = NUM_SAMPLES // _NTILES


def _sc_gather_body(w_hbm, b_hbm, tgt_hbm, smp_hbm,
                    tw_out, sw_out, tb_out, sb_out,
                    idxt_v, idxs_v, trows_v, srows_v, tbias_v, sbias_v, sem):
    c = lax.axis_index("c")
    s = lax.axis_index("s")
    wid = s * _NC + c
    tbase = wid * _TGT_PER
    sbase = wid * _SMP_PER
    pltpu.sync_copy(tgt_hbm.at[pl.ds(tbase, _TGT_PER)], idxt_v)
    pltpu.sync_copy(smp_hbm.at[pl.ds(sbase, _SMP_PER)], idxs_v)
    cps = [
        pltpu.async_copy(w_hbm.at[idxt_v], trows_v, sem),
        pltpu.async_copy(w_hbm.at[idxs_v], srows_v, sem),
        pltpu.async_copy(b_hbm.at[idxt_v], tbias_v, sem),
        pltpu.async_copy(b_hbm.at[idxs_v], sbias_v, sem),
    ]
    for cp in cps:
        cp.wait()
    pltpu.sync_copy(trows_v, tw_out.at[pl.ds(tbase, _TGT_PER)])
    pltpu.sync_copy(srows_v, sw_out.at[pl.ds(sbase, _SMP_PER)])
    pltpu.sync_copy(tbias_v, tb_out.at[pl.ds(tbase, _TGT_PER)])
    pltpu.sync_copy(sbias_v, sb_out.at[pl.ds(sbase, _SMP_PER)])


def _sc_gather(softmax_w, softmax_b, targets, sampled_ids):
    mesh = plsc.VectorSubcoreMesh(core_axis_name="c", subcore_axis_name="s")
    f = pl.kernel(
        _sc_gather_body,
        out_type=(
            jax.ShapeDtypeStruct((BATCH, EMB_DIM), jnp.float32),
            jax.ShapeDtypeStruct((NUM_SAMPLES, EMB_DIM), jnp.float32),
            jax.ShapeDtypeStruct((BATCH,), jnp.float32),
            jax.ShapeDtypeStruct((NUM_SAMPLES,), jnp.float32),
        ),
        mesh=mesh,
        scratch_types=(
            pltpu.VMEM((_TGT_PER,), jnp.int32),
            pltpu.VMEM((_SMP_PER,), jnp.int32),
            pltpu.VMEM((_TGT_PER, EMB_DIM), jnp.float32),
            pltpu.VMEM((_SMP_PER, EMB_DIM), jnp.float32),
            pltpu.VMEM((_TGT_PER,), jnp.float32),
            pltpu.VMEM((_SMP_PER,), jnp.float32),
            pltpu.SemaphoreType.DMA,
        ),
    )
    return f(softmax_w, softmax_b, targets, sampled_ids)


_VOCAB_PAD = 100096


_ZSL = _VOCAB_PAD // _NS
_SMP_PER_SC = NUM_SAMPLES // _NS
_TGT_PER_SC = BATCH // _NTILES


def _sc_member_body(zeros_hbm, ones_hbm, smp_hbm, tgt_hbm,
                    flags_out, b0_hbm, b1_hbm,
                    zbuf, idx_v, val_v, fval_v, sem):
    c = lax.axis_index("c")
    s = lax.axis_index("s")
    zbase = s * _ZSL
    pltpu.sync_copy(zeros_hbm.at[pl.ds(0, _ZSL)], zbuf)

    @pl.when(c == 0)
    def _():
        pltpu.sync_copy(zbuf, b0_hbm.at[pl.ds(zbase, _ZSL)])

    @pl.when(c == 1)
    def _():
        pltpu.sync_copy(zbuf, b1_hbm.at[pl.ds(zbase, _ZSL)])

    plsc.subcore_barrier()

    sbase = s * _SMP_PER_SC
    pltpu.sync_copy(smp_hbm.at[pl.ds(sbase, _SMP_PER_SC)], idx_v)
    pltpu.sync_copy(ones_hbm, val_v)

    @pl.when(c == 0)
    def _():
        pltpu.async_copy(val_v, b0_hbm.at[idx_v], sem).wait()

    @pl.when(c == 1)
    def _():
        pltpu.async_copy(val_v, b1_hbm.at[idx_v], sem).wait()

    plsc.subcore_barrier()

    tbase = c * (BATCH // _NC) + s * _TGT_PER_SC
    pltpu.sync_copy(tgt_hbm.at[pl.ds(tbase, _TGT_PER_SC)], idx_v)

    @pl.when(c == 0)
    def _():
        pltpu.async_copy(b0_hbm.at[idx_v], fval_v, sem).wait()

    @pl.when(c == 1)
    def _():
        pltpu.async_copy(b1_hbm.at[idx_v], fval_v, sem).wait()

    pltpu.sync_copy(fval_v, flags_out.at[pl.ds(tbase, _TGT_PER_SC)])


def _sc_member(sampled_ids, targets):
    mesh = plsc.VectorSubcoreMesh(core_axis_name="c", subcore_axis_name="s")
    f = pl.kernel(
        _sc_member_body,
        out_type=(
            jax.ShapeDtypeStruct((BATCH,), jnp.float32),
            jax.ShapeDtypeStruct((_VOCAB_PAD,), jnp.float32),
            jax.ShapeDtypeStruct((_VOCAB_PAD,), jnp.float32),
        ),
        mesh=mesh,
        scratch_types=(
            pltpu.VMEM((_ZSL,), jnp.float32),
            pltpu.VMEM((_SMP_PER_SC,), jnp.int32),
            pltpu.VMEM((_SMP_PER_SC,), jnp.float32),
            pltpu.VMEM((_TGT_PER_SC,), jnp.float32),
            pltpu.SemaphoreType.DMA,
        ),
    )
    zeros = jnp.zeros((_VOCAB_PAD,), jnp.float32)
    ones = jnp.ones((_SMP_PER_SC,), jnp.float32)
    flags, _, _ = f(zeros, ones, sampled_ids, targets)
    return flags


def _tc_loss_body(nt_ref, emb_ref, tw_ref, sw_ref, tb_ref, sb_ref,
                  tgt_ref, sid_ref, out_ref):
    step = pl.program_id(0)
    nt = nt_ref[0, 0]

    e2 = emb_ref[...] * LOG2E
    t_dot2 = jnp.sum(e2 * tw_ref[...], axis=1, keepdims=True)
    tgtf = tgt_ref[...].astype(jnp.float32)
    tp = jnp.log((tgtf + 2.0) / (tgtf + 1.0)) * INV_LOG_NW
    tec = 1.0 - jnp.exp(nt * jnp.log(1.0 - tp))
    t_logit2 = t_dot2 + tb_ref[...] * LOG2E - jnp.log2(tec + TINY)

    sidf = sid_ref[...].astype(jnp.float32)
    sp = jnp.log((sidf + 2.0) / (sidf + 1.0)) * INV_LOG_NW
    sec = 1.0 - jnp.exp(nt * jnp.log(1.0 - sp))
    adj2 = sb_ref[...] * LOG2E - jnp.log2(sec + TINY)

    logits2 = lax.dot_general(e2, sw_ref[...], (((1,), (1,)), ((), ())),
                              precision=lax.Precision.DEFAULT,
                              preferred_element_type=jnp.float32)
    expl = jnp.exp2(logits2 + adj2)
    expl = jnp.where(sid_ref[...] == tgt_ref[...], 0.0, expl)
    ssum = jnp.sum(expl, axis=1, keepdims=True)
    den = ssum + jnp.exp2(t_logit2)
    partial = jnp.sum(jnp.log2(den) - t_logit2) * LN2

    @pl.when(step == 0)
    def _():
        out_ref[0, 0] = 0.0

    out_ref[0, 0] += partial


def _tc_loss(nt, emb, tw, sw, tb, sb, tgt, sid, interpret=False):
    grid = (BATCH // BM,)
    return pl.pallas_call(
        _tc_loss_body,
        grid=grid,
        in_specs=[
            pl.BlockSpec(memory_space=pltpu.SMEM),
            pl.BlockSpec((BM, EMB_DIM), lambda i: (i, 0)),
            pl.BlockSpec((BM, EMB_DIM), lambda i: (i, 0)),
            pl.BlockSpec((NUM_SAMPLES, EMB_DIM), lambda i: (0, 0)),
            pl.BlockSpec((BM, 1), lambda i: (i, 0)),
            pl.BlockSpec((1, NUM_SAMPLES), lambda i: (0, 0)),
            pl.BlockSpec((BM, 1), lambda i: (i, 0)),
            pl.BlockSpec((1, NUM_SAMPLES), lambda i: (0, 0)),
        ],
        out_specs=pl.BlockSpec(memory_space=pltpu.SMEM),
        out_shape=jax.ShapeDtypeStruct((1, 1), jnp.float32),
        interpret=interpret,
    )(nt, emb, tw, sw, tb, sb, tgt, sid)


def kernel(embeddings, targets, softmax_w, softmax_b, sampled_ids, num_tries):
    tw, sw, tb, sb = _sc_gather(softmax_w, softmax_b, targets, sampled_ids)
    nt = jnp.asarray(num_tries, jnp.float32).reshape(1, 1)
    loss = _tc_loss(
        nt, embeddings, tw, sw,
        tb.reshape(BATCH, 1), sb.reshape(1, NUM_SAMPLES),
        targets.reshape(BATCH, 1), sampled_ids.reshape(1, NUM_SAMPLES),
    )
    return loss[0, 0]


# speedup vs baseline: 1.1817x; 1.0622x over previous
import numpy as np
import jax
import jax.numpy as jnp
from jax import lax
from jax.experimental import pallas as pl
from jax.experimental.pallas import tpu as pltpu
from jax.experimental.pallas import tpu_sc as plsc

NUM_WORDS = 100000
EMB_DIM = 128
NUM_SAMPLES = 8192
BATCH = 16384
TINY = 1e-13
INV_LOG_NW = float(1.0 / np.log(NUM_WORDS + 1))
LOG2E = float(np.log2(np.e))
LN2 = float(np.log(2.0))

BM = 2048

_NC = 2
_NS = 16
_NTILES = _NC * _NS
_TGT_PER = BATCH // _NTILES
_SMP_PER = NUM_SAMPLES // _NTILES


def _sc_gather_body(w_hbm, b_hbm, tgt_hbm, smp_hbm,
                    tw_out, sw_out, tb_out, sb_out,
                    idxt_v, idxs_v, trows_v, srows_v, tbias_v, sbias_v, sem):
    c = lax.axis_index("c")
    s = lax.axis_index("s")
    wid = s * _NC + c
    tbase = wid * _TGT_PER
    sbase = wid * _SMP_PER
    pltpu.sync_copy(tgt_hbm.at[pl.ds(tbase, _TGT_PER)], idxt_v)
    pltpu.sync_copy(smp_hbm.at[pl.ds(sbase, _SMP_PER)], idxs_v)
    cps = [
        pltpu.async_copy(w_hbm.at[idxt_v], trows_v, sem),
        pltpu.async_copy(w_hbm.at[idxs_v], srows_v, sem),
        pltpu.async_copy(b_hbm.at[idxt_v], tbias_v, sem),
        pltpu.async_copy(b_hbm.at[idxs_v], sbias_v, sem),
    ]
    for cp in cps:
        cp.wait()
    pltpu.sync_copy(trows_v, tw_out.at[pl.ds(tbase, _TGT_PER)])
    pltpu.sync_copy(srows_v, sw_out.at[pl.ds(sbase, _SMP_PER)])
    pltpu.sync_copy(tbias_v, tb_out.at[pl.ds(tbase, _TGT_PER)])
    pltpu.sync_copy(sbias_v, sb_out.at[pl.ds(sbase, _SMP_PER)])


def _sc_gather(softmax_w, softmax_b, targets, sampled_ids):
    mesh = plsc.VectorSubcoreMesh(core_axis_name="c", subcore_axis_name="s")
    f = pl.kernel(
        _sc_gather_body,
        out_type=(
            jax.ShapeDtypeStruct((BATCH, EMB_DIM), jnp.float32),
            jax.ShapeDtypeStruct((NUM_SAMPLES, EMB_DIM), jnp.float32),
            jax.ShapeDtypeStruct((BATCH,), jnp.float32),
            jax.ShapeDtypeStruct((NUM_SAMPLES,), jnp.float32),
        ),
        mesh=mesh,
        scratch_types=(
            pltpu.VMEM((_TGT_PER,), jnp.int32),
            pltpu.VMEM((_SMP_PER,), jnp.int32),
            pltpu.VMEM((_TGT_PER, EMB_DIM), jnp.float32),
            pltpu.VMEM((_SMP_PER, EMB_DIM), jnp.float32),
            pltpu.VMEM((_TGT_PER,), jnp.float32),
            pltpu.VMEM((_SMP_PER,), jnp.float32),
            pltpu.SemaphoreType.DMA,
        ),
    )
    return f(softmax_w, softmax_b, targets, sampled_ids)


_VOCAB_PAD = 100096


_ZSL = _VOCAB_PAD // _NS
_SMP_PER_SC = NUM_SAMPLES // _NS
_TGT_PER_SC = BATCH // _NTILES


def _sc_member_body(zeros_hbm, ones_hbm, smp_hbm, tgt_hbm,
                    flags_out, b0_hbm, b1_hbm,
                    zbuf, idx_v, val_v, fval_v, sem):
    c = lax.axis_index("c")
    s = lax.axis_index("s")
    zbase = s * _ZSL
    pltpu.sync_copy(zeros_hbm.at[pl.ds(0, _ZSL)], zbuf)

    @pl.when(c == 0)
    def _():
        pltpu.sync_copy(zbuf, b0_hbm.at[pl.ds(zbase, _ZSL)])

    @pl.when(c == 1)
    def _():
        pltpu.sync_copy(zbuf, b1_hbm.at[pl.ds(zbase, _ZSL)])

    plsc.subcore_barrier()

    sbase = s * _SMP_PER_SC
    pltpu.sync_copy(smp_hbm.at[pl.ds(sbase, _SMP_PER_SC)], idx_v)
    pltpu.sync_copy(ones_hbm, val_v)

    @pl.when(c == 0)
    def _():
        pltpu.async_copy(val_v, b0_hbm.at[idx_v], sem).wait()

    @pl.when(c == 1)
    def _():
        pltpu.async_copy(val_v, b1_hbm.at[idx_v], sem).wait()

    plsc.subcore_barrier()

    tbase = c * (BATCH // _NC) + s * _TGT_PER_SC
    pltpu.sync_copy(tgt_hbm.at[pl.ds(tbase, _TGT_PER_SC)], idx_v)

    @pl.when(c == 0)
    def _():
        pltpu.async_copy(b0_hbm.at[idx_v], fval_v, sem).wait()

    @pl.when(c == 1)
    def _():
        pltpu.async_copy(b1_hbm.at[idx_v], fval_v, sem).wait()

    pltpu.sync_copy(fval_v, flags_out.at[pl.ds(tbase, _TGT_PER_SC)])


def _sc_member(sampled_ids, targets):
    mesh = plsc.VectorSubcoreMesh(core_axis_name="c", subcore_axis_name="s")
    f = pl.kernel(
        _sc_member_body,
        out_type=(
            jax.ShapeDtypeStruct((BATCH,), jnp.float32),
            jax.ShapeDtypeStruct((_VOCAB_PAD,), jnp.float32),
            jax.ShapeDtypeStruct((_VOCAB_PAD,), jnp.float32),
        ),
        mesh=mesh,
        scratch_types=(
            pltpu.VMEM((_ZSL,), jnp.float32),
            pltpu.VMEM((_SMP_PER_SC,), jnp.int32),
            pltpu.VMEM((_SMP_PER_SC,), jnp.float32),
            pltpu.VMEM((_TGT_PER_SC,), jnp.float32),
            pltpu.SemaphoreType.DMA,
        ),
    )
    zeros = jnp.zeros((_VOCAB_PAD,), jnp.float32)
    ones = jnp.ones((_SMP_PER_SC,), jnp.float32)
    flags, _, _ = f(zeros, ones, sampled_ids, targets)
    return flags


def _tc_loss_body(nt_ref, emb_ref, tw_ref, sw_ref, tb_ref, sb_ref,
                  tgt_ref, sid_ref, out_ref):
    step = pl.program_id(0)
    nt = nt_ref[0, 0]

    e2 = emb_ref[...] * LOG2E
    t_dot2 = jnp.sum(e2 * tw_ref[...], axis=1, keepdims=True)
    tgtf = tgt_ref[...].astype(jnp.float32)
    tp = jnp.log((tgtf + 2.0) / (tgtf + 1.0)) * INV_LOG_NW
    tec = 1.0 - jnp.exp(nt * jnp.log(1.0 - tp))
    t_logit2 = t_dot2 + tb_ref[...] * LOG2E - jnp.log2(tec + TINY)

    sidf = sid_ref[...].astype(jnp.float32)
    sp = jnp.log((sidf + 2.0) / (sidf + 1.0)) * INV_LOG_NW
    sec = 1.0 - jnp.exp(nt * jnp.log(1.0 - sp))
    adj2 = sb_ref[...] * LOG2E - jnp.log2(sec + TINY)

    logits2 = lax.dot_general(e2, sw_ref[...], (((1,), (1,)), ((), ())),
                              precision=lax.Precision.DEFAULT,
                              preferred_element_type=jnp.float32)
    expl = jnp.exp2(logits2 + adj2)
    expl = jnp.where(sid_ref[...] == tgt_ref[...], 0.0, expl)
    ssum = jnp.sum(expl, axis=1, keepdims=True)
    den = ssum + jnp.exp2(t_logit2)
    partial = jnp.sum(jnp.log2(den) - t_logit2) * LN2

    @pl.when(step == 0)
    def _():
        out_ref[0, 0] = 0.0

    out_ref[0, 0] += partial


def _tc_loss(nt, emb, tw, sw, tb, sb, tgt, sid, interpret=False):
    grid = (BATCH // BM,)
    return pl.pallas_call(
        _tc_loss_body,
        grid=grid,
        in_specs=[
            pl.BlockSpec(memory_space=pltpu.SMEM),
            pl.BlockSpec((BM, EMB_DIM), lambda i: (i, 0)),
            pl.BlockSpec((BM, EMB_DIM), lambda i: (i, 0)),
            pl.BlockSpec((NUM_SAMPLES, EMB_DIM), lambda i: (0, 0)),
            pl.BlockSpec((BM, 1), lambda i: (i, 0)),
            pl.BlockSpec((1, NUM_SAMPLES), lambda i: (0, 0)),
            pl.BlockSpec((BM, 1), lambda i: (i, 0)),
            pl.BlockSpec((1, NUM_SAMPLES), lambda i: (0, 0)),
        ],
        out_specs=pl.BlockSpec(memory_space=pltpu.SMEM),
        out_shape=jax.ShapeDtypeStruct((1, 1), jnp.float32),
        interpret=interpret,
    )(nt, emb, tw, sw, tb, sb, tgt, sid)


def kernel(embeddings, targets, softmax_w, softmax_b, sampled_ids, num_tries):
    tw, sw, tb, sb = _sc_gather(softmax_w, softmax_b, targets, sampled_ids)
    nt = jnp.asarray(num_tries, jnp.float32).reshape(1, 1)
    loss = _tc_loss(
        nt, embeddings, tw, sw,
        tb.reshape(BATCH, 1), sb.reshape(1, NUM_SAMPLES),
        targets.reshape(BATCH, 1), sampled_ids.reshape(1, NUM_SAMPLES),
    )
    return loss[0, 0]


# speedup vs baseline: 1.2571x; 1.0638x over previous
import numpy as np
import jax
import jax.numpy as jnp
from jax import lax
from jax.experimental import pallas as pl
from jax.experimental.pallas import tpu as pltpu
from jax.experimental.pallas import tpu_sc as plsc

NUM_WORDS = 100000
EMB_DIM = 128
NUM_SAMPLES = 8192
BATCH = 16384
TINY = 1e-13
INV_LOG_NW = float(1.0 / np.log(NUM_WORDS + 1))
LOG2E = float(np.log2(np.e))
LN2 = float(np.log(2.0))

BM = 4096

_NC = 2
_NS = 16
_NTILES = _NC * _NS
_TGT_PER = BATCH // _NTILES
_SMP_PER = NUM_SAMPLES // _NTILES


def _sc_gather_body(w_hbm, b_hbm, tgt_hbm, smp_hbm,
                    tw_out, sw_out, tb_out, sb_out,
                    idxt_v, idxs_v, trows_v, srows_v, tbias_v, sbias_v, sem):
    c = lax.axis_index("c")
    s = lax.axis_index("s")
    wid = s * _NC + c
    tbase = wid * _TGT_PER
    sbase = wid * _SMP_PER
    pltpu.sync_copy(tgt_hbm.at[pl.ds(tbase, _TGT_PER)], idxt_v)
    pltpu.sync_copy(smp_hbm.at[pl.ds(sbase, _SMP_PER)], idxs_v)
    cps = [
        pltpu.async_copy(w_hbm.at[idxt_v], trows_v, sem),
        pltpu.async_copy(w_hbm.at[idxs_v], srows_v, sem),
        pltpu.async_copy(b_hbm.at[idxt_v], tbias_v, sem),
        pltpu.async_copy(b_hbm.at[idxs_v], sbias_v, sem),
    ]
    for cp in cps:
        cp.wait()
    pltpu.sync_copy(trows_v, tw_out.at[pl.ds(tbase, _TGT_PER)])
    pltpu.sync_copy(srows_v, sw_out.at[pl.ds(sbase, _SMP_PER)])
    pltpu.sync_copy(tbias_v, tb_out.at[pl.ds(tbase, _TGT_PER)])
    pltpu.sync_copy(sbias_v, sb_out.at[pl.ds(sbase, _SMP_PER)])


def _sc_gather(softmax_w, softmax_b, targets, sampled_ids):
    mesh = plsc.VectorSubcoreMesh(core_axis_name="c", subcore_axis_name="s")
    f = pl.kernel(
        _sc_gather_body,
        out_type=(
            jax.ShapeDtypeStruct((BATCH, EMB_DIM), jnp.float32),
            jax.ShapeDtypeStruct((NUM_SAMPLES, EMB_DIM), jnp.float32),
            jax.ShapeDtypeStruct((BATCH,), jnp.float32),
            jax.ShapeDtypeStruct((NUM_SAMPLES,), jnp.float32),
        ),
        mesh=mesh,
        scratch_types=(
            pltpu.VMEM((_TGT_PER,), jnp.int32),
            pltpu.VMEM((_SMP_PER,), jnp.int32),
            pltpu.VMEM((_TGT_PER, EMB_DIM), jnp.float32),
            pltpu.VMEM((_SMP_PER, EMB_DIM), jnp.float32),
            pltpu.VMEM((_TGT_PER,), jnp.float32),
            pltpu.VMEM((_SMP_PER,), jnp.float32),
            pltpu.SemaphoreType.DMA,
        ),
    )
    return f(softmax_w, softmax_b, targets, sampled_ids)


_VOCAB_PAD = 100096


_ZSL = _VOCAB_PAD // _NS
_SMP_PER_SC = NUM_SAMPLES // _NS
_TGT_PER_SC = BATCH // _NTILES


def _sc_member_body(zeros_hbm, ones_hbm, smp_hbm, tgt_hbm,
                    flags_out, b0_hbm, b1_hbm,
                    zbuf, idx_v, val_v, fval_v, sem):
    c = lax.axis_index("c")
    s = lax.axis_index("s")
    zbase = s * _ZSL
    pltpu.sync_copy(zeros_hbm.at[pl.ds(0, _ZSL)], zbuf)

    @pl.when(c == 0)
    def _():
        pltpu.sync_copy(zbuf, b0_hbm.at[pl.ds(zbase, _ZSL)])

    @pl.when(c == 1)
    def _():
        pltpu.sync_copy(zbuf, b1_hbm.at[pl.ds(zbase, _ZSL)])

    plsc.subcore_barrier()

    sbase = s * _SMP_PER_SC
    pltpu.sync_copy(smp_hbm.at[pl.ds(sbase, _SMP_PER_SC)], idx_v)
    pltpu.sync_copy(ones_hbm, val_v)

    @pl.when(c == 0)
    def _():
        pltpu.async_copy(val_v, b0_hbm.at[idx_v], sem).wait()

    @pl.when(c == 1)
    def _():
        pltpu.async_copy(val_v, b1_hbm.at[idx_v], sem).wait()

    plsc.subcore_barrier()

    tbase = c * (BATCH // _NC) + s * _TGT_PER_SC
    pltpu.sync_copy(tgt_hbm.at[pl.ds(tbase, _TGT_PER_SC)], idx_v)

    @pl.when(c == 0)
    def _():
        pltpu.async_copy(b0_hbm.at[idx_v], fval_v, sem).wait()

    @pl.when(c == 1)
    def _():
        pltpu.async_copy(b1_hbm.at[idx_v], fval_v, sem).wait()

    pltpu.sync_copy(fval_v, flags_out.at[pl.ds(tbase, _TGT_PER_SC)])


def _sc_member(sampled_ids, targets):
    mesh = plsc.VectorSubcoreMesh(core_axis_name="c", subcore_axis_name="s")
    f = pl.kernel(
        _sc_member_body,
        out_type=(
            jax.ShapeDtypeStruct((BATCH,), jnp.float32),
            jax.ShapeDtypeStruct((_VOCAB_PAD,), jnp.float32),
            jax.ShapeDtypeStruct((_VOCAB_PAD,), jnp.float32),
        ),
        mesh=mesh,
        scratch_types=(
            pltpu.VMEM((_ZSL,), jnp.float32),
            pltpu.VMEM((_SMP_PER_SC,), jnp.int32),
            pltpu.VMEM((_SMP_PER_SC,), jnp.float32),
            pltpu.VMEM((_TGT_PER_SC,), jnp.float32),
            pltpu.SemaphoreType.DMA,
        ),
    )
    zeros = jnp.zeros((_VOCAB_PAD,), jnp.float32)
    ones = jnp.ones((_SMP_PER_SC,), jnp.float32)
    flags, _, _ = f(zeros, ones, sampled_ids, targets)
    return flags


def _tc_loss_body(nt_ref, emb_ref, tw_ref, sw_ref, tb_ref, sb_ref,
                  tgt_ref, sid_ref, out_ref):
    step = pl.program_id(0)
    nt = nt_ref[0, 0]

    e2 = emb_ref[...] * LOG2E
    t_dot2 = jnp.sum(e2 * tw_ref[...], axis=1, keepdims=True)
    tgtf = tgt_ref[...].astype(jnp.float32)
    tp = jnp.log((tgtf + 2.0) / (tgtf + 1.0)) * INV_LOG_NW
    tec = 1.0 - jnp.exp(nt * jnp.log(1.0 - tp))
    t_logit2 = t_dot2 + tb_ref[...] * LOG2E - jnp.log2(tec + TINY)

    sidf = sid_ref[...].astype(jnp.float32)
    sp = jnp.log((sidf + 2.0) / (sidf + 1.0)) * INV_LOG_NW
    sec = 1.0 - jnp.exp(nt * jnp.log(1.0 - sp))
    adj2 = sb_ref[...] * LOG2E - jnp.log2(sec + TINY)

    logits2 = lax.dot_general(e2, sw_ref[...], (((1,), (1,)), ((), ())),
                              precision=lax.Precision.DEFAULT,
                              preferred_element_type=jnp.float32)
    expl = jnp.exp2(logits2 + adj2)
    expl = jnp.where(sid_ref[...] == tgt_ref[...], 0.0, expl)
    ssum = jnp.sum(expl, axis=1, keepdims=True)
    den = ssum + jnp.exp2(t_logit2)
    partial = jnp.sum(jnp.log2(den) - t_logit2) * LN2

    @pl.when(step == 0)
    def _():
        out_ref[0, 0] = 0.0

    out_ref[0, 0] += partial


def _tc_loss(nt, emb, tw, sw, tb, sb, tgt, sid, interpret=False):
    grid = (BATCH // BM,)
    return pl.pallas_call(
        _tc_loss_body,
        grid=grid,
        in_specs=[
            pl.BlockSpec(memory_space=pltpu.SMEM),
            pl.BlockSpec((BM, EMB_DIM), lambda i: (i, 0)),
            pl.BlockSpec((BM, EMB_DIM), lambda i: (i, 0)),
            pl.BlockSpec((NUM_SAMPLES, EMB_DIM), lambda i: (0, 0)),
            pl.BlockSpec((BM, 1), lambda i: (i, 0)),
            pl.BlockSpec((1, NUM_SAMPLES), lambda i: (0, 0)),
            pl.BlockSpec((BM, 1), lambda i: (i, 0)),
            pl.BlockSpec((1, NUM_SAMPLES), lambda i: (0, 0)),
        ],
        out_specs=pl.BlockSpec(memory_space=pltpu.SMEM),
        out_shape=jax.ShapeDtypeStruct((1, 1), jnp.float32),
        interpret=interpret,
    )(nt, emb, tw, sw, tb, sb, tgt, sid)


def kernel(embeddings, targets, softmax_w, softmax_b, sampled_ids, num_tries):
    tw, sw, tb, sb = _sc_gather(softmax_w, softmax_b, targets, sampled_ids)
    nt = jnp.asarray(num_tries, jnp.float32).reshape(1, 1)
    loss = _tc_loss(
        nt, embeddings, tw, sw,
        tb.reshape(BATCH, 1), sb.reshape(1, NUM_SAMPLES),
        targets.reshape(BATCH, 1), sampled_ids.reshape(1, NUM_SAMPLES),
    )
    return loss[0, 0]
